# baseline (speedup 1.0000x reference)
import functools

import jax
import jax.numpy as jnp
from jax import lax
from jax.experimental import pallas as pl
from jax.experimental.pallas import tpu as pltpu
from jax.experimental.pallas import tpu_sc as plsc

NC = 2
NS = 16
L = 16
TOTAL = 32768
D = 256
BATCH = 16
NW = NC * NS
TOKW = TOTAL // NW
CHUNK_S = 128
CHUNK_V = 64
NV = D // L
NSC = 4
NVEC = (TOKW - NSC * CHUNK_S) // CHUNK_V
VBASE = NSC * CHUNK_S


def _sc_body(flat_hbm, cu_hbm, out_hbm,
             cu_v, sbuf0, sbuf1, vbuf0, vbuf1, idx0, idx1, pubidx,
             acc_lo, acc_hi, comb, row, shared_lo, shared_hi, cu_sm,
             ssem00, ssem01, ssem10, ssem11, vsem0, vsem1,
             scsem00, scsem01, scsem10, scsem11):
    c = lax.axis_index("c")
    s = lax.axis_index("s")
    tok0 = (c * NS + s) * TOKW
    slot0 = s * BATCH

    pltpu.sync_copy(cu_hbm.at[pl.ds(0, L)], cu_v)
    cuv = cu_v[...]
    iota = lax.iota(jnp.int32, L)
    cu_splats = [jnp.take(cuv, jnp.full((L,), b, jnp.int32))
                 for b in range(1, BATCH)]
    cu_sm[0] = jnp.int32(0)
    for b in range(1, BATCH):
        cu_sm[b] = jnp.squeeze(lax.slice(cuv, (b,), (b + 1,)))
    cu_sm[BATCH] = jnp.int32(TOTAL)

    one = jnp.ones((L,), jnp.int32)
    zero = jnp.zeros((L,), jnp.int32)
    fzero = jnp.zeros((L,), jnp.float32)

    for b in range(BATCH):
        for j in range(NV):
            comb[b, pl.ds(j * L, L)] = fzero
            if j < NV // 2:
                acc_lo[b, pl.ds(j * L, L)] = fzero
            else:
                acc_hi[b, pl.ds((j - NV // 2) * L, L)] = fzero
    pltpu.sync_copy(acc_lo, shared_lo.at[pl.ds(slot0, BATCH)])
    pltpu.sync_copy(acc_hi, shared_hi.at[pl.ds(slot0, BATCH)])

    sbufs, idxbs = [sbuf0, sbuf1], [idx0, idx1]
    ssems = [[ssem00, ssem01], [ssem10, ssem11]]
    scsems = [[scsem00, scsem01], [scsem10, scsem11]]
    vbufs, vsems = [vbuf0, vbuf1], [vsem0, vsem1]
    sc_cps = [None] * NSC

    def in_copy_s(i, buf, sems):
        return [
            pltpu.async_copy(
                flat_hbm.at[pl.ds(tok0 + i * CHUNK_S, CHUNK_S),
                            pl.ds(h * (D // 2), D // 2)],
                buf.at[h], sems[h])
            for h in range(2)
        ]

    def in_copy_v(m, buf, sem):
        return pltpu.async_copy(
            flat_hbm.at[pl.ds(tok0 + VBASE + m * CHUNK_V, CHUNK_V)], buf, sem)

    in_s = [None] * max(NSC, 1)
    in_v = [None] * max(NVEC, 1)
    if NSC > 0:
        in_s[0] = in_copy_s(0, sbufs[0], ssems[0])
    if NVEC > 0:
        in_v[0] = in_copy_v(0, vbufs[0], vsems[0])

    def do_scatter_item(i):
        if i >= 1:
            sc_cps[i - 1][0].wait()
            sc_cps[i - 1][1].wait()
        if i + 1 < NSC:
            in_s[i + 1] = in_copy_s(i + 1, sbufs[(i + 1) % 2],
                                    ssems[(i + 1) % 2])
        in_s[i][0].wait()
        in_s[i][1].wait()
        t0 = tok0 + i * CHUNK_S
        ib = idxbs[i % 2]
        for g in range(CHUNK_S // L):
            tvec = iota + (t0 + g * L)
            seg = zero
            for cs_ in cu_splats:
                seg = seg + jnp.where(tvec >= cs_, one, zero)
            ib[pl.ds(g * L, L)] = seg + slot0
        sc_cps[i] = [
            pltpu.async_copy(sbufs[i % 2].at[0], shared_lo.at[ib],
                             scsems[i % 2][0], add=True),
            pltpu.async_copy(sbufs[i % 2].at[1], shared_hi.at[ib],
                             scsems[i % 2][1], add=True),
        ]

    def do_vector_item(m):
        if m + 1 < NVEC:
            in_v[m + 1] = in_copy_v(m + 1, vbufs[(m + 1) % 2],
                                    vsems[(m + 1) % 2])
        in_v[m].wait()
        buf = vbufs[m % 2]
        t0 = tok0 + VBASE + m * CHUNK_V

        def seg_body(b, carry):
            lo = cu_sm[b]
            hi = cu_sm[b + 1]
            start = jnp.clip(lo - t0, 0, CHUNK_V)
            end = jnp.clip(hi - t0, 0, CHUNK_V)

            @pl.when(end > start)
            def _():
                def tok_body(t, accs):
                    return tuple(accs[j] + buf[t, pl.ds(j * L, L)]
                                 for j in range(NV))
                accs = lax.fori_loop(start, end, tok_body,
                                     tuple(fzero for _ in range(NV)))
                for j in range(NV // 2):
                    plsc.addupdate(acc_lo.at[b, pl.ds(j * L, L)], accs[j])
                for j in range(NV // 2, NV):
                    plsc.addupdate(
                        acc_hi.at[b, pl.ds((j - NV // 2) * L, L)], accs[j])
            return carry

        lax.fori_loop(0, BATCH, seg_body, 0)

    si, vi = 0, 0
    for _ in range(NSC + NVEC):
        if si < NSC and (vi >= NVEC or si * 2 <= vi):
            do_scatter_item(si)
            si += 1
        else:
            do_vector_item(vi)
            vi += 1

    if NSC > 0:
        sc_cps[NSC - 1][0].wait()
        sc_cps[NSC - 1][1].wait()

    pubidx[pl.ds(0, L)] = iota + slot0
    pltpu.sync_copy(acc_lo, shared_lo.at[pubidx], add=True)
    pltpu.sync_copy(acc_hi, shared_hi.at[pubidx], add=True)

    plsc.subcore_barrier()

    for i in range(NS):
        pltpu.sync_copy(shared_lo.at[i * BATCH + s],
                        comb.at[i, pl.ds(0, D // 2)])
        pltpu.sync_copy(shared_hi.at[i * BATCH + s],
                        comb.at[i, pl.ds(D // 2, D // 2)])
    for j in range(NV):
        r = comb[0, pl.ds(j * L, L)]
        for i in range(1, NS):
            r = r + comb[i, pl.ds(j * L, L)]
        row[pl.ds(j * L, L)] = r
    pltpu.sync_copy(row, out_hbm.at[c, s])


_mesh = plsc.VectorSubcoreMesh(core_axis_name="c", subcore_axis_name="s")

_sc_kernel = functools.partial(
    pl.kernel,
    out_type=jax.ShapeDtypeStruct((NC, BATCH, D), jnp.float32),
    mesh=_mesh,
    scratch_types=[
        pltpu.VMEM((L,), jnp.int32),
        pltpu.VMEM((2, CHUNK_S, D // 2), jnp.float32),
        pltpu.VMEM((2, CHUNK_S, D // 2), jnp.float32),
        pltpu.VMEM((CHUNK_V, D), jnp.float32),
        pltpu.VMEM((CHUNK_V, D), jnp.float32),
        pltpu.VMEM((CHUNK_S,), jnp.int32),
        pltpu.VMEM((CHUNK_S,), jnp.int32),
        pltpu.VMEM((L,), jnp.int32),
        pltpu.VMEM((BATCH, D // 2), jnp.float32),
        pltpu.VMEM((BATCH, D // 2), jnp.float32),
        pltpu.VMEM((NS, D), jnp.float32),
        pltpu.VMEM((D,), jnp.float32),
        pltpu.VMEM_SHARED((NS * BATCH, D // 2), jnp.float32),
        pltpu.VMEM_SHARED((NS * BATCH, D // 2), jnp.float32),
        pltpu.SMEM((BATCH + 1,), jnp.int32),
        pltpu.SemaphoreType.DMA,
        pltpu.SemaphoreType.DMA,
        pltpu.SemaphoreType.DMA,
        pltpu.SemaphoreType.DMA,
        pltpu.SemaphoreType.DMA,
        pltpu.SemaphoreType.DMA,
        pltpu.SemaphoreType.DMA,
        pltpu.SemaphoreType.DMA,
        pltpu.SemaphoreType.DMA,
        pltpu.SemaphoreType.DMA,
    ],
)(_sc_body)


def _tc_fold(p_ref, o_ref):
    o_ref[...] = p_ref[0] + p_ref[1]


_tc_combine = pl.pallas_call(
    _tc_fold,
    out_shape=jax.ShapeDtypeStruct((BATCH, D), jnp.float32),
)


@jax.jit
def kernel(flat, cu_seqlens):
    partials = _sc_kernel(flat, cu_seqlens)
    return _tc_combine(partials)


# speedup vs baseline: 1.1391x; 1.1391x over previous
import functools

import jax
import jax.numpy as jnp
from jax import lax
from jax.experimental import pallas as pl
from jax.experimental.pallas import tpu as pltpu
from jax.experimental.pallas import tpu_sc as plsc

NC = 2
NS = 16
L = 16
TOTAL = 32768
D = 256
BATCH = 16
DC = D // NC
TOK = TOTAL // NS
CHUNK = 128
NCHUNK = TOK // CHUNK
NG = CHUNK // L
NV = DC // L
NSC = 4
NVEC = NCHUNK - NSC


def _sc_body(flat_hbm, cu_hbm, out_hbm,
             cu_v, sbuf0, sbuf1, vbuf0, vbuf1, idx0, idx1, pubidx,
             acc, comb, row, shared, cu_sm,
             ssem0, ssem1, vsem0, vsem1, scsem0, scsem1):
    c = lax.axis_index("c")
    s = lax.axis_index("s")
    col0 = c * DC
    tok0 = s * TOK
    slot0 = s * BATCH

    pltpu.sync_copy(cu_hbm.at[pl.ds(0, L)], cu_v)
    cuv = cu_v[...]
    iota = lax.iota(jnp.int32, L)
    cu_splats = [jnp.take(cuv, jnp.full((L,), b, jnp.int32))
                 for b in range(1, BATCH)]
    cu_sm[0] = jnp.int32(0)
    for b in range(1, BATCH):
        cu_sm[b] = jnp.squeeze(lax.slice(cuv, (b,), (b + 1,)))
    cu_sm[BATCH] = jnp.int32(TOTAL)

    one = jnp.ones((L,), jnp.int32)
    zero = jnp.zeros((L,), jnp.int32)
    fzero = jnp.zeros((L,), jnp.float32)

    for b in range(BATCH):
        for j in range(NV):
            comb[b, pl.ds(j * L, L)] = fzero
            acc[b, pl.ds(j * L, L)] = fzero
    pltpu.sync_copy(comb, shared.at[pl.ds(slot0, BATCH)])

    sbufs, ssems, scsems, idxbs = [sbuf0, sbuf1], [ssem0, ssem1], \
        [scsem0, scsem1], [idx0, idx1]
    vbufs, vsems = [vbuf0, vbuf1], [vsem0, vsem1]
    sc_cps = [None] * NSC

    def in_copy(k, buf, sem):
        return pltpu.async_copy(
            flat_hbm.at[pl.ds(tok0 + k * CHUNK, CHUNK), pl.ds(col0, DC)],
            buf, sem)

    in_s = [None] * NSC
    in_v = [None] * NVEC
    if NSC > 0:
        in_s[0] = in_copy(0, sbufs[0], ssems[0])
    in_v[0] = in_copy(NSC, vbufs[0], vsems[0])

    def do_scatter_item(i):
        if i >= 1:
            sc_cps[i - 1].wait()
        if i + 1 < NSC:
            in_s[i + 1] = in_copy(i + 1, sbufs[(i + 1) % 2],
                                  ssems[(i + 1) % 2])
        in_s[i].wait()
        t0 = tok0 + i * CHUNK
        ib = idxbs[i % 2]
        for g in range(NG):
            tvec = iota + (t0 + g * L)
            seg = zero
            for cs_ in cu_splats:
                seg = seg + jnp.where(tvec >= cs_, one, zero)
            ib[pl.ds(g * L, L)] = seg + slot0
        sc_cps[i] = pltpu.async_copy(sbufs[i % 2], shared.at[ib],
                                     scsems[i % 2], add=True)

    def do_vector_item(m):
        if m + 1 < NVEC:
            in_v[m + 1] = in_copy(NSC + m + 1, vbufs[(m + 1) % 2],
                                  vsems[(m + 1) % 2])
        in_v[m].wait()
        buf = vbufs[m % 2]
        t0 = tok0 + (NSC + m) * CHUNK

        def seg_body(b, carry):
            lo = cu_sm[b]
            hi = cu_sm[b + 1]
            start = jnp.clip(lo - t0, 0, CHUNK)
            end = jnp.clip(hi - t0, 0, CHUNK)

            @pl.when(end > start)
            def _():
                n = end - start
                half = n // 2

                def tok2_body(i2, accs):
                    t = start + i2 * 2
                    accs = tuple(accs[j] + buf[t, pl.ds(j * L, L)]
                                 for j in range(NV))
                    return tuple(accs[j] + buf[t + 1, pl.ds(j * L, L)]
                                 for j in range(NV))
                accs = lax.fori_loop(0, half, tok2_body,
                                     tuple(fzero for _ in range(NV)))
                for j in range(NV):
                    plsc.addupdate(acc.at[b, pl.ds(j * L, L)], accs[j])

                @pl.when(n != half * 2)
                def _():
                    for j in range(NV):
                        plsc.addupdate(acc.at[b, pl.ds(j * L, L)],
                                       buf[end - 1, pl.ds(j * L, L)])
            return carry

        lax.fori_loop(0, BATCH, seg_body, 0)

    si, vi = 0, 0
    for _ in range(NCHUNK):
        if si < NSC and (vi >= NVEC or si <= vi):
            do_scatter_item(si)
            si += 1
        else:
            do_vector_item(vi)
            vi += 1

    if NSC > 0:
        sc_cps[NSC - 1].wait()

    pubidx[pl.ds(0, L)] = iota + slot0
    pltpu.sync_copy(acc, shared.at[pubidx], add=True)

    plsc.subcore_barrier()

    for i in range(NS):
        pltpu.sync_copy(shared.at[i * BATCH + s], comb.at[i])
    for j in range(NV):
        r = comb[0, pl.ds(j * L, L)]
        for i in range(1, NS):
            r = r + comb[i, pl.ds(j * L, L)]
        row[pl.ds(j * L, L)] = r
    pltpu.sync_copy(row, out_hbm.at[s, pl.ds(col0, DC)])


_mesh = plsc.VectorSubcoreMesh(core_axis_name="c", subcore_axis_name="s")

_sc_kernel = functools.partial(
    pl.kernel,
    out_type=jax.ShapeDtypeStruct((BATCH, D), jnp.float32),
    mesh=_mesh,
    scratch_types=[
        pltpu.VMEM((L,), jnp.int32),
        pltpu.VMEM((CHUNK, DC), jnp.float32),
        pltpu.VMEM((CHUNK, DC), jnp.float32),
        pltpu.VMEM((CHUNK, DC), jnp.float32),
        pltpu.VMEM((CHUNK, DC), jnp.float32),
        pltpu.VMEM((CHUNK,), jnp.int32),
        pltpu.VMEM((CHUNK,), jnp.int32),
        pltpu.VMEM((L,), jnp.int32),
        pltpu.VMEM((BATCH, DC), jnp.float32),
        pltpu.VMEM((NS, DC), jnp.float32),
        pltpu.VMEM((DC,), jnp.float32),
        pltpu.VMEM_SHARED((NS * BATCH, DC), jnp.float32),
        pltpu.SMEM((BATCH + 1,), jnp.int32),
        pltpu.SemaphoreType.DMA,
        pltpu.SemaphoreType.DMA,
        pltpu.SemaphoreType.DMA,
        pltpu.SemaphoreType.DMA,
        pltpu.SemaphoreType.DMA,
        pltpu.SemaphoreType.DMA,
    ],
)(_sc_body)


@jax.jit
def kernel(flat, cu_seqlens):
    return _sc_kernel(flat, cu_seqlens)


# speedup vs baseline: 1.1685x; 1.0258x over previous
import functools

import jax
import jax.numpy as jnp
from jax import lax
from jax.experimental import pallas as pl
from jax.experimental.pallas import tpu as pltpu
from jax.experimental.pallas import tpu_sc as plsc

NC = 2
NS = 16
L = 16
TOTAL = 32768
D = 256
BATCH = 16
DC = D // NC
TOK = TOTAL // NS
CHUNK = 128
NCHUNK = TOK // CHUNK
NG = CHUNK // L
NV = DC // L
NSC = 4
NVEC = NCHUNK - NSC


def _sc_body(flat_hbm, cu_hbm, out_hbm,
             cu_v, sbuf0, sbuf1, vbuf0, vbuf1, idx0, idx1, pubidx,
             acc, comb, row, shared, cu_sm,
             ssem0, ssem1, vsem0, vsem1, scsem0, scsem1):
    c = lax.axis_index("c")
    s = lax.axis_index("s")
    col0 = c * DC
    tok0 = s * TOK
    slot0 = s * BATCH

    pltpu.sync_copy(cu_hbm.at[pl.ds(0, L)], cu_v)
    cuv = cu_v[...]
    iota = lax.iota(jnp.int32, L)
    cu_splats = [jnp.take(cuv, jnp.full((L,), b, jnp.int32))
                 for b in range(1, BATCH)]
    cu_sm[0] = jnp.int32(0)
    for b in range(1, BATCH):
        cu_sm[b] = jnp.squeeze(lax.slice(cuv, (b,), (b + 1,)))
    cu_sm[BATCH] = jnp.int32(TOTAL)

    one = jnp.ones((L,), jnp.int32)
    zero = jnp.zeros((L,), jnp.int32)
    fzero = jnp.zeros((L,), jnp.float32)

    for b in range(BATCH):
        for j in range(NV):
            comb[b, pl.ds(j * L, L)] = fzero
            acc[b, pl.ds(j * L, L)] = fzero
    pltpu.sync_copy(comb, shared.at[pl.ds(slot0, BATCH)])

    sbufs, ssems, scsems, idxbs = [sbuf0, sbuf1], [ssem0, ssem1], \
        [scsem0, scsem1], [idx0, idx1]
    vbufs, vsems = [vbuf0, vbuf1], [vsem0, vsem1]
    sc_cps = [None] * NSC

    def in_copy(k, buf, sem):
        return pltpu.async_copy(
            flat_hbm.at[pl.ds(tok0 + k * CHUNK, CHUNK), pl.ds(col0, DC)],
            buf, sem)

    in_s = [None] * NSC
    in_v = [None] * NVEC
    if NSC > 0:
        in_s[0] = in_copy(0, sbufs[0], ssems[0])
    in_v[0] = in_copy(NSC, vbufs[0], vsems[0])

    def do_scatter_item(i):
        if i >= 1:
            sc_cps[i - 1].wait()
        if i + 1 < NSC:
            in_s[i + 1] = in_copy(i + 1, sbufs[(i + 1) % 2],
                                  ssems[(i + 1) % 2])
        in_s[i].wait()
        t0 = tok0 + i * CHUNK
        ib = idxbs[i % 2]
        for g in range(NG):
            tvec = iota + (t0 + g * L)
            seg = zero
            for cs_ in cu_splats:
                seg = seg + jnp.where(tvec >= cs_, one, zero)
            ib[pl.ds(g * L, L)] = seg + slot0
        sc_cps[i] = pltpu.async_copy(sbufs[i % 2], shared.at[ib],
                                     scsems[i % 2], add=True)

    def do_vector_item(m):
        if m + 1 < NVEC:
            in_v[m + 1] = in_copy(NSC + m + 1, vbufs[(m + 1) % 2],
                                  vsems[(m + 1) % 2])
        in_v[m].wait()
        buf = vbufs[m % 2]
        t0 = tok0 + (NSC + m) * CHUNK

        def seg_body(b, carry):
            lo = cu_sm[b]
            hi = cu_sm[b + 1]
            start = jnp.clip(lo - t0, 0, CHUNK)
            end = jnp.clip(hi - t0, 0, CHUNK)

            @pl.when(end > start)
            def _():
                def tok_body(t, accs):
                    return tuple(accs[j] + buf[t, pl.ds(j * L, L)]
                                 for j in range(NV))
                accs = lax.fori_loop(start, end, tok_body,
                                     tuple(fzero for _ in range(NV)))
                for j in range(NV):
                    plsc.addupdate(acc.at[b, pl.ds(j * L, L)], accs[j])
            return carry

        lax.fori_loop(0, BATCH, seg_body, 0)

    si, vi = 0, 0
    for _ in range(NCHUNK):
        if si < NSC and (vi >= NVEC or si <= vi):
            do_scatter_item(si)
            si += 1
        else:
            do_vector_item(vi)
            vi += 1

    if NSC > 0:
        sc_cps[NSC - 1].wait()

    pubidx[pl.ds(0, L)] = iota + slot0
    pltpu.sync_copy(acc, shared.at[pubidx], add=True)

    plsc.subcore_barrier()

    for i in range(NS):
        pltpu.sync_copy(shared.at[i * BATCH + s], comb.at[i])
    for j in range(NV):
        r = comb[0, pl.ds(j * L, L)]
        for i in range(1, NS):
            r = r + comb[i, pl.ds(j * L, L)]
        row[pl.ds(j * L, L)] = r
    pltpu.sync_copy(row, out_hbm.at[s, pl.ds(col0, DC)])


_mesh = plsc.VectorSubcoreMesh(core_axis_name="c", subcore_axis_name="s")

_sc_kernel = functools.partial(
    pl.kernel,
    out_type=jax.ShapeDtypeStruct((BATCH, D), jnp.float32),
    mesh=_mesh,
    scratch_types=[
        pltpu.VMEM((L,), jnp.int32),
        pltpu.VMEM((CHUNK, DC), jnp.float32),
        pltpu.VMEM((CHUNK, DC), jnp.float32),
        pltpu.VMEM((CHUNK, DC), jnp.float32),
        pltpu.VMEM((CHUNK, DC), jnp.float32),
        pltpu.VMEM((CHUNK,), jnp.int32),
        pltpu.VMEM((CHUNK,), jnp.int32),
        pltpu.VMEM((L,), jnp.int32),
        pltpu.VMEM((BATCH, DC), jnp.float32),
        pltpu.VMEM((NS, DC), jnp.float32),
        pltpu.VMEM((DC,), jnp.float32),
        pltpu.VMEM_SHARED((NS * BATCH, DC), jnp.float32),
        pltpu.SMEM((BATCH + 1,), jnp.int32),
        pltpu.SemaphoreType.DMA,
        pltpu.SemaphoreType.DMA,
        pltpu.SemaphoreType.DMA,
        pltpu.SemaphoreType.DMA,
        pltpu.SemaphoreType.DMA,
        pltpu.SemaphoreType.DMA,
    ],
)(_sc_body)


@jax.jit
def kernel(flat, cu_seqlens):
    return _sc_kernel(flat, cu_seqlens)
